# baseline (speedup 1.0000x reference)
import functools

import jax
import jax.numpy as jnp
from jax import lax
from jax.experimental import pallas as pl
from jax.experimental.pallas import tpu as pltpu
from jax.experimental.pallas import tpu_sc as plsc

B = 16384
V = 100000
D = 64
BASE_DIM = D * 5 + D // 2

NW = 32
CHUNK = 4096
NCHUNK = B // CHUNK
FB = 32

_TABLES = (
    (0, 0, 0, D),
    (1, 1, D, D),
    (2, 2, 2 * D, D),
    (4, 4, 3 * D + D // 2, D),
    (5, 1, 4 * D + D // 2, D),
    (3, 3, 3 * D, D // 2),
)


def _make_sc_gather(acol, ncols):
    ncpw = ncols // NW
    mesh = plsc.VectorSubcoreMesh(core_axis_name="c", subcore_axis_name="s")

    @functools.partial(
        pl.kernel,
        mesh=mesh,
        out_type=jax.ShapeDtypeStruct((ncols * B,), jnp.float32),
        scratch_types=[
            pltpu.VMEM((V,), jnp.float32),
            pltpu.VMEM((B,), jnp.int32),
            pltpu.VMEM((2, CHUNK), jnp.float32),
            pltpu.SemaphoreType.DMA,
        ],
        compiler_params=pltpu.CompilerParams(
            use_tc_tiling_on_sc=False, needs_layout_passes=False),
    )
    def k(act_hbm, tbl, out_hbm, col_v, idx_v, chunk_v, sem):
        wid = lax.axis_index("s") * 2 + lax.axis_index("c")
        H = V // 2
        hidx = pltpu.async_copy(act_hbm.at[acol, :], idx_v, sem)
        d0 = wid * ncpw
        hc0 = pltpu.async_copy(tbl.at[d0, pl.ds(0, H)], col_v.at[pl.ds(0, H)],
                               sem)
        hc1 = pltpu.async_copy(tbl.at[d0, pl.ds(H, H)], col_v.at[pl.ds(H, H)],
                               sem)
        hidx.wait()
        for kk in range(ncpw):
            d = wid * ncpw + kk
            if kk == 0:
                hc0.wait()
                hc1.wait()
            else:
                h0 = pltpu.async_copy(
                    tbl.at[d, pl.ds(0, H)], col_v.at[pl.ds(0, H)], sem)
                h1 = pltpu.async_copy(
                    tbl.at[d, pl.ds(H, H)], col_v.at[pl.ds(H, H)], sem)
                h0.wait()
                h1.wait()
            handles = []
            for ci in range(NCHUNK):
                b = ci % 2
                if ci >= 2:
                    handles[ci - 2].wait()

                def body(j, carry):
                    src = idx_v[pl.ds(ci * CHUNK + j * 16, 16)]
                    chunk_v[b, pl.ds(j * 16, 16)] = plsc.load_gather(
                        col_v, [src])
                    return carry

                lax.fori_loop(0, CHUNK // 16, body, 0, unroll=16)
                handles.append(pltpu.async_copy(
                    chunk_v.at[b],
                    out_hbm.at[pl.ds(d * B + ci * CHUNK, CHUNK)], sem))
            handles[-2].wait()
            handles[-1].wait()

    return k


def _bn_compute(x_ref, g_ref, b_ref, y_ref):
    x = x_ref[...]
    xb = x.astype(jnp.bfloat16).astype(jnp.float32)
    m = jnp.mean(xb, axis=(1, 2), keepdims=True)
    dlt = xb - m
    v = jnp.mean(dlt * dlt, axis=(1, 2), keepdims=True)
    inv = lax.rsqrt(v + 1e-5)
    gg = g_ref[...].reshape(FB, 1, 1)
    bb = b_ref[...].reshape(FB, 1, 1)
    y_ref[...] = (dlt * (inv * gg) + bb).reshape(FB, B)


def _bn_body_first(x_ref, g_ref, b_ref, y_ref):
    _bn_compute(x_ref, g_ref, b_ref, y_ref)


def _bn_body(x_ref, g_ref, b_ref, _, y_ref):
    _bn_compute(x_ref, g_ref, b_ref, y_ref)


def _bn_bias_body(x_ref, g_ref, b_ref, _, y_ref, bias_ref):
    bias_ref[...] = x_ref[...].reshape(FB, B)
    _bn_compute(x_ref, g_ref, b_ref, y_ref)


def _tc_batchnorm_step(x1d, gamma, beta, y_prev, foff, ncols, with_bias):
    nblk = ncols // FB
    x3 = x1d.reshape(ncols, B // 128, 128)
    g2 = gamma[foff:foff + ncols].reshape(ncols, 1)
    b2 = beta[foff:foff + ncols].reshape(ncols, 1)
    base = foff // FB
    out_shape = [jax.ShapeDtypeStruct((BASE_DIM, B), jnp.float32)]
    out_specs = [pl.BlockSpec((FB, B), lambda i: (base + i, 0))]
    if with_bias:
        out_shape.append(jax.ShapeDtypeStruct((ncols, B), jnp.float32))
        out_specs.append(pl.BlockSpec((FB, B), lambda i: (i, 0)))
    in_specs = [
        pl.BlockSpec((FB, B // 128, 128), lambda i: (i, 0, 0)),
        pl.BlockSpec((FB, 1), lambda i: (i, 0)),
        pl.BlockSpec((FB, 1), lambda i: (i, 0)),
    ]
    args = [x3, g2, b2]
    aliases = {}
    if y_prev is not None:
        in_specs.append(pl.BlockSpec(memory_space=pl.ANY))
        args.append(y_prev)
        aliases = {3: 0}
        body = _bn_bias_body if with_bias else _bn_body
    else:
        body = _bn_body_first
    return pl.pallas_call(
        body,
        grid=(nblk,),
        in_specs=in_specs,
        out_specs=out_specs,
        out_shape=out_shape,
        input_output_aliases=aliases,
    )(*args)


def kernel(action, pre_embed, W0, W1, W2, W3, W4, bn_gamma, bn_beta):
    actT = action.T
    tablesT = (W0.T, W1.T, W2.T, W3.T, W4.T, pre_embed.T)
    gathered = {}
    for (ti, acol, foff, ncols) in _TABLES:
        gathered[ti] = _make_sc_gather(acol, ncols)(actT, tablesT[ti])

    y = None
    biasT = None
    for (ti, acol, foff, ncols) in _TABLES:
        res = _tc_batchnorm_step(
            gathered[ti], bn_gamma, bn_beta, y, foff, ncols, ti == 2)
        if ti == 2:
            y, biasT = res
        else:
            (y,) = res
    return (y.T, biasT.T)


# speedup vs baseline: 1.0165x; 1.0165x over previous
import functools

import jax
import jax.numpy as jnp
from jax import lax
from jax.experimental import pallas as pl
from jax.experimental.pallas import tpu as pltpu
from jax.experimental.pallas import tpu_sc as plsc

B = 16384
V = 100000
D = 64
BASE_DIM = D * 5 + D // 2

NW = 32
CHUNK = 2048
NCHUNK = B // CHUNK
FB = 32

_TABLES = (
    (0, 0, 0, D),
    (1, 1, D, D),
    (2, 2, 2 * D, D),
    (4, 4, 3 * D + D // 2, D),
    (5, 1, 4 * D + D // 2, D),
    (3, 3, 3 * D, D // 2),
)


def _make_sc_gather(acol, ncols):
    ncpw = ncols // NW
    mesh = plsc.VectorSubcoreMesh(core_axis_name="c", subcore_axis_name="s")

    @functools.partial(
        pl.kernel,
        mesh=mesh,
        out_type=jax.ShapeDtypeStruct((ncols * B,), jnp.float32),
        scratch_types=[
            pltpu.VMEM((V,), jnp.float32),
            pltpu.VMEM((B,), jnp.int32),
            pltpu.VMEM((2, CHUNK), jnp.float32),
            pltpu.SemaphoreType.DMA,
        ],
        compiler_params=pltpu.CompilerParams(
            use_tc_tiling_on_sc=False, needs_layout_passes=False),
    )
    def k(act_hbm, tbl, out_hbm, col_v, idx_v, chunk_v, sem):
        wid = lax.axis_index("s") * 2 + lax.axis_index("c")
        H = V // 2
        hidx = pltpu.async_copy(act_hbm.at[acol, :], idx_v, sem)
        d0 = wid * ncpw
        hc0 = pltpu.async_copy(tbl.at[d0, pl.ds(0, H)], col_v.at[pl.ds(0, H)],
                               sem)
        hc1 = pltpu.async_copy(tbl.at[d0, pl.ds(H, H)], col_v.at[pl.ds(H, H)],
                               sem)
        hidx.wait()
        for kk in range(ncpw):
            d = wid * ncpw + kk
            if kk == 0:
                hc0.wait()
                hc1.wait()
            else:
                h0 = pltpu.async_copy(
                    tbl.at[d, pl.ds(0, H)], col_v.at[pl.ds(0, H)], sem)
                h1 = pltpu.async_copy(
                    tbl.at[d, pl.ds(H, H)], col_v.at[pl.ds(H, H)], sem)
                h0.wait()
                h1.wait()
            handles = []
            for ci in range(NCHUNK):
                b = ci % 2
                if ci >= 2:
                    handles[ci - 2].wait()

                def body(j, carry):
                    src = idx_v[pl.ds(ci * CHUNK + j * 16, 16)]
                    chunk_v[b, pl.ds(j * 16, 16)] = plsc.load_gather(
                        col_v, [src])
                    return carry

                lax.fori_loop(0, CHUNK // 16, body, 0, unroll=16)
                handles.append(pltpu.async_copy(
                    chunk_v.at[b],
                    out_hbm.at[pl.ds(d * B + ci * CHUNK, CHUNK)], sem))
            handles[-2].wait()
            handles[-1].wait()

    return k


def _bn_compute(x_ref, g_ref, b_ref, y_ref):
    x = x_ref[...]
    xb = x.astype(jnp.bfloat16).astype(jnp.float32)
    m = jnp.mean(xb, axis=(1, 2), keepdims=True)
    dlt = xb - m
    v = jnp.mean(dlt * dlt, axis=(1, 2), keepdims=True)
    inv = lax.rsqrt(v + 1e-5)
    gg = g_ref[...].reshape(FB, 1, 1)
    bb = b_ref[...].reshape(FB, 1, 1)
    y_ref[...] = (dlt * (inv * gg) + bb).reshape(FB, B)


def _bn_body_first(x_ref, g_ref, b_ref, y_ref):
    _bn_compute(x_ref, g_ref, b_ref, y_ref)


def _bn_body(x_ref, g_ref, b_ref, _, y_ref):
    _bn_compute(x_ref, g_ref, b_ref, y_ref)


def _bn_bias_body(x_ref, g_ref, b_ref, _, y_ref, bias_ref):
    bias_ref[...] = x_ref[...].reshape(FB, B)
    _bn_compute(x_ref, g_ref, b_ref, y_ref)


def _tc_batchnorm_step(x1d, gamma, beta, y_prev, foff, ncols, with_bias):
    nblk = ncols // FB
    x3 = x1d.reshape(ncols, B // 128, 128)
    g2 = gamma[foff:foff + ncols].reshape(ncols, 1)
    b2 = beta[foff:foff + ncols].reshape(ncols, 1)
    base = foff // FB
    out_shape = [jax.ShapeDtypeStruct((BASE_DIM, B), jnp.float32)]
    out_specs = [pl.BlockSpec((FB, B), lambda i: (base + i, 0))]
    if with_bias:
        out_shape.append(jax.ShapeDtypeStruct((ncols, B), jnp.float32))
        out_specs.append(pl.BlockSpec((FB, B), lambda i: (i, 0)))
    in_specs = [
        pl.BlockSpec((FB, B // 128, 128), lambda i: (i, 0, 0)),
        pl.BlockSpec((FB, 1), lambda i: (i, 0)),
        pl.BlockSpec((FB, 1), lambda i: (i, 0)),
    ]
    args = [x3, g2, b2]
    aliases = {}
    if y_prev is not None:
        in_specs.append(pl.BlockSpec(memory_space=pl.ANY))
        args.append(y_prev)
        aliases = {3: 0}
        body = _bn_bias_body if with_bias else _bn_body
    else:
        body = _bn_body_first
    return pl.pallas_call(
        body,
        grid=(nblk,),
        in_specs=in_specs,
        out_specs=out_specs,
        out_shape=out_shape,
        input_output_aliases=aliases,
    )(*args)


def kernel(action, pre_embed, W0, W1, W2, W3, W4, bn_gamma, bn_beta):
    actT = action.T
    tablesT = (W0.T, W1.T, W2.T, W3.T, W4.T, pre_embed.T)
    gathered = {}
    for (ti, acol, foff, ncols) in _TABLES:
        gathered[ti] = _make_sc_gather(acol, ncols)(actT, tablesT[ti])

    y = None
    biasT = None
    for (ti, acol, foff, ncols) in _TABLES:
        res = _tc_batchnorm_step(
            gathered[ti], bn_gamma, bn_beta, y, foff, ncols, ti == 2)
        if ti == 2:
            y, biasT = res
        else:
            (y,) = res
    return (y.T, biasT.T)
